# Initial kernel scaffold; baseline (speedup 1.0000x reference)
import jax
import jax.numpy as jnp
from jax.experimental import pallas as pl


def kernel(adj_indices, adj_values, adj2_indices, adj2_values, emb_node, emb_attri, W1, W2):
    raise NotImplementedError("write your pallas kernel here")


# speedup vs baseline: 3.8133x; 3.8133x over previous
import functools

import jax
import jax.numpy as jnp
from jax import lax
from jax.experimental import pallas as pl
from jax.experimental.pallas import tpu as pltpu
from jax.experimental.pallas import tpu_sc as plsc

NNODE = 8000
NATTRI = 2000
N = NNODE + NATTRI
F = 128
E = 320000

NC = 2
NS = 16
CHUNK = 128
NCHUNK = 160
GRP = 32
NGRP = NCHUNK // GRP
EPT_PAD = NCHUNK * CHUNK
NPAD = 10240
ROWS_PER_TILE = NPAD // NS
ZROWS = 128
NZ = ROWS_PER_TILE // ZROWS


def _prep_adj(adj_indices, adj_values):
    dst = adj_indices[0]
    src = adj_indices[1]
    pad = NS * EPT_PAD - E
    zi = jnp.zeros((pad,), jnp.int32)
    src = jnp.concatenate([src, zi]).reshape(NS, NCHUNK, CHUNK)
    dst = jnp.concatenate([dst, zi]).reshape(NS, NCHUNK, CHUNK)
    val = jnp.concatenate([adj_values, jnp.zeros((pad,), jnp.float32)])
    val = val.reshape(NS, NCHUNK, CHUNK)
    return src, dst, val


_RB = 1000


def _tc_body(node_ref, attri_ref, w1_ref, w2_ref, x1_ref, y_ref):
    i = pl.program_id(0)
    x = jnp.where(i < 8, node_ref[...], attri_ref[...])
    x1_ref[...] = x
    y_ref[0] = jnp.dot(x, w1_ref[...], preferred_element_type=jnp.float32)
    y_ref[1] = jnp.dot(x, w2_ref[...], preferred_element_type=jnp.float32)


def _tc_call(emb_node, emb_attri, W1, W2):
    return pl.pallas_call(
        _tc_body,
        grid=(N // _RB,),
        in_specs=[
            pl.BlockSpec((_RB, F), lambda i: (jnp.minimum(i, 7), 0)),
            pl.BlockSpec((_RB, F), lambda i: (jnp.maximum(i - 8, 0), 0)),
            pl.BlockSpec((F, F), lambda i: (0, 0)),
            pl.BlockSpec((F, F), lambda i: (0, 0)),
        ],
        out_specs=[
            pl.BlockSpec((_RB, F), lambda i: (i, 0)),
            pl.BlockSpec((2, _RB, F), lambda i: (0, i, 0)),
        ],
        out_shape=[
            jax.ShapeDtypeStruct((N, F), jnp.float32),
            jax.ShapeDtypeStruct((2, N, F), jnp.float32),
        ],
    )(emb_node, emb_attri, W1, W2)


_GATHER_DNUMS = lax.GatherDimensionNumbers(
    offset_dims=(), collapsed_slice_dims=(0,), start_index_map=(0,))


def _lane_broadcast(v16, r):
    idx = jnp.full((16, 1), r, jnp.int32)
    return lax.gather(v16, idx, _GATHER_DNUMS, (1,),
                      mode=lax.GatherScatterMode.PROMISE_IN_BOUNDS)

_sc_mesh = plsc.VectorSubcoreMesh(
    core_axis_name="c", subcore_axis_name="s", num_cores=NC, num_subcores=NS
)


@functools.partial(
    pl.kernel,
    out_type=jax.ShapeDtypeStruct((NC, NPAD, F), jnp.float32),
    mesh=_sc_mesh,
    scratch_types=[
        pltpu.VMEM((GRP, CHUNK), jnp.int32),
        pltpu.VMEM((GRP, CHUNK), jnp.int32),
        pltpu.VMEM((GRP, CHUNK), jnp.float32),
        pltpu.VMEM((CHUNK, F), jnp.float32),
        pltpu.VMEM_SHARED((NPAD, F), jnp.float32),
        pltpu.SemaphoreType.DMA,
    ],
)
def _sc_body(y_hbm, src_hbm, dst_hbm, val_hbm, out_hbm,
             src_v, dst_v, val_v, rows_v, acc, sem):
    c = lax.axis_index("c")
    s = lax.axis_index("s")

    zero = jnp.zeros((16,), jnp.float32)

    def zrow(r, carry):
        for k in range(F // 16):
            rows_v[r, pl.ds(k * 16, 16)] = zero
        return carry

    lax.fori_loop(0, ZROWS, zrow, 0)
    base = s * ROWS_PER_TILE
    for k in range(NZ):
        pltpu.sync_copy(rows_v, acc.at[pl.ds(base + k * ZROWS, ZROWS)])
    plsc.subcore_barrier()

    def scale_chunk(j):
        def group(g, carry2):
            v16 = val_v[j, pl.ds(g * 16, 16)]
            for r in range(16):
                bc = _lane_broadcast(v16, r)
                row = g * 16 + r
                for k in range(F // 16):
                    rows_v[row, pl.ds(k * 16, 16)] = (
                        rows_v[row, pl.ds(k * 16, 16)] * bc)
            return carry2

        lax.fori_loop(0, CHUNK // 16, group, 0)

    for grp in range(NGRP):
        pltpu.sync_copy(src_hbm.at[c, s, pl.ds(grp * GRP, GRP)], src_v)
        pltpu.sync_copy(dst_hbm.at[c, s, pl.ds(grp * GRP, GRP)], dst_v)
        pltpu.sync_copy(val_hbm.at[c, s, pl.ds(grp * GRP, GRP)], val_v)

        def chunk(j, carry):
            pltpu.async_copy(y_hbm.at[c].at[src_v.at[j]], rows_v, sem).wait()
            scale_chunk(j)
            pltpu.sync_copy(rows_v, acc.at[dst_v.at[j]], add=True)
            return carry

        lax.fori_loop(0, GRP, chunk, 0)
    plsc.subcore_barrier()

    for k2 in range(NZ):
        pltpu.sync_copy(acc.at[pl.ds(base + k2 * ZROWS, ZROWS)], rows_v)

        def rrow(r, carry):
            for k in range(F // 16):
                v = rows_v[r, pl.ds(k * 16, 16)]
                rows_v[r, pl.ds(k * 16, 16)] = jnp.maximum(v, 0.0)
            return carry

        lax.fori_loop(0, ZROWS, rrow, 0)
        pltpu.sync_copy(rows_v, out_hbm.at[c, pl.ds(base + k2 * ZROWS, ZROWS)])


def kernel(adj_indices, adj_values, adj2_indices, adj2_values,
           emb_node, emb_attri, W1, W2):
    src1, dst1, val1 = _prep_adj(adj_indices, adj_values)
    src2, dst2, val2 = _prep_adj(adj2_indices, adj2_values)
    src = jnp.stack([src1, src2])
    dst = jnp.stack([dst1, dst2])
    val = jnp.stack([val1, val2])
    x1, y = _tc_call(emb_node, emb_attri, W1, W2)
    out = _sc_body(y, src, dst, val)
    return (x1, out[0, :N], out[1, :N])
